# baseline (speedup 1.0000x reference)
import functools

import jax
import jax.numpy as jnp
from jax import lax
from jax.experimental import pallas as pl
from jax.experimental.pallas import tpu as pltpu
from jax.experimental.pallas import tpu_sc as plsc

_F = 128
_L = 100
_DP = 1024
_DH = _DP // 2
_NSUB = 16
_B_SC = 512
_GROUP = 16


def _build_m_body(keys_ref, level_ref, mlo_ref, mhi_ref):
    m = (keys_ref[...][:, None, :] * level_ref[...][None, :, :]).reshape(
        8 * _L, _DP)
    mlo_ref[...] = m[:, :_DH]
    mhi_ref[...] = m[:, _DH:]


def _build_m(keys_p, level_p):
    return pl.pallas_call(
        _build_m_body,
        grid=(_F // 8,),
        in_specs=[
            pl.BlockSpec((8, _DP), lambda g: (g, 0)),
            pl.BlockSpec((_L, _DP), lambda g: (0, 0)),
        ],
        out_specs=[
            pl.BlockSpec((8 * _L, _DH), lambda g: (g, 0)),
            pl.BlockSpec((8 * _L, _DH), lambda g: (g, 0)),
        ],
        out_shape=[
            jax.ShapeDtypeStruct((_F * _L, _DH), jnp.float32),
            jax.ShapeDtypeStruct((_F * _L, _DH), jnp.float32),
        ],
    )(keys_p, level_p)


def _round_half_even_idx(s):
    x = (s + 1.0) * 49.5
    f0 = x.astype(jnp.int32)
    rf = x - f0.astype(jnp.float32)
    odd = (f0 & 1) == 1
    up = (rf > 0.5) | ((rf == 0.5) & odd)
    q = f0 + jnp.where(up, 1, 0)
    return jnp.minimum(jnp.maximum(q, 0), _L - 1)


def _make_sc_bag(rpt):
    mesh = plsc.VectorSubcoreMesh(
        core_axis_name="c", subcore_axis_name="s", num_cores=2,
        num_subcores=_NSUB)

    @functools.partial(
        pl.kernel,
        out_type=jax.ShapeDtypeStruct((2, _NSUB, rpt, 2, 16), jnp.float32),
        mesh=mesh,
        scratch_types=[
            pltpu.VMEM((rpt, _F), jnp.float32),
            pltpu.VMEM((rpt, _F), jnp.int32),
            pltpu.VMEM((_F, _DH), jnp.float32),
            pltpu.VMEM((2, _DH), jnp.float32),
            pltpu.VMEM((rpt, 2, 16), jnp.float32),
            pltpu.SemaphoreType.DMA,
        ],
    )
    def sc_bag(mlo_hbm, mhi_hbm, samples_hbm, cent_hbm, out_hbm,
               samp_v, j_v, rows_v, cent_v, out_v, sem):
        core = lax.axis_index("c")
        sub = lax.axis_index("s")
        pltpu.sync_copy(samples_hbm.at[sub], samp_v)
        pltpu.sync_copy(cent_hbm.at[core], cent_v)
        iota16 = lax.iota(jnp.int32, 16)

        def quant_body(i, carry):
            for c in range(_F // 16):
                q = _round_half_even_idx(samp_v[i, pl.ds(c * 16, 16)])
                j_v[i, pl.ds(c * 16, 16)] = (c * 16 + iota16) * _L + q
            return carry
        lax.fori_loop(0, rpt, quant_body, 0)

        def row_body(i, carry):
            @pl.when(core == 0)
            def _():
                pltpu.async_copy(mlo_hbm.at[j_v.at[i]], rows_v, sem).wait()

            @pl.when(core == 1)
            def _():
                pltpu.async_copy(mhi_hbm.at[j_v.at[i]], rows_v, sem).wait()

            def acc_body(r, accs):
                return tuple(a + rows_v[r, pl.ds(k * 16, 16)]
                             for k, a in enumerate(accs))
            acc0 = tuple(rows_v[0, pl.ds(k * 16, 16)] for k in range(_DH // 16))
            accs = lax.fori_loop(1, _F, acc_body, acc0)

            s0 = jnp.zeros((16,), jnp.float32)
            s1 = jnp.zeros((16,), jnp.float32)
            for k in range(_DH // 16):
                enc = jnp.sign(accs[k])
                s0 = s0 + enc * cent_v[0, pl.ds(k * 16, 16)]
                s1 = s1 + enc * cent_v[1, pl.ds(k * 16, 16)]
            out_v[i, 0] = s0
            out_v[i, 1] = s1
            return carry
        lax.fori_loop(0, rpt, row_body, 0)
        pltpu.sync_copy(out_v, out_hbm.at[core, sub])

    return sc_bag


def _tc_body(samples_ref, keys_ref, level_ref, cent_ref, out_ref):
    B = samples_ref.shape[0]
    x = (samples_ref[...] + 1.0) * (0.5 * (_L - 1))
    idx = jnp.clip(jnp.round(x), 0, _L - 1).astype(jnp.int32)
    level = level_ref[...].astype(jnp.bfloat16)
    l_iota = lax.broadcasted_iota(jnp.int32, (1, _GROUP, _L), 2)
    acc = jnp.zeros((B, _DP), jnp.float32)
    for g in range(_F // _GROUP):
        keys_g = keys_ref[pl.ds(g * _GROUP, _GROUP), :].astype(jnp.bfloat16)
        m_g = (keys_g[:, None, :] * level[None, :, :]).reshape(_GROUP * _L, _DP)
        idx_g = idx[:, g * _GROUP:(g + 1) * _GROUP]
        oh = (idx_g[:, :, None] == l_iota).astype(jnp.bfloat16).reshape(
            B, _GROUP * _L)
        acc = acc + jnp.dot(oh, m_g, preferred_element_type=jnp.float32)
    enc = jnp.sign(acc)
    out_ref[...] = lax.dot_general(
        enc, cent_ref[...], (((1,), (1,)), ((), ())),
        preferred_element_type=jnp.float32)


def _tc_slice(samples_tc, keys_p, level_p, cent_p):
    B = samples_tc.shape[0]
    return pl.pallas_call(
        _tc_body,
        out_shape=jax.ShapeDtypeStruct((B, cent_p.shape[0]), jnp.float32),
    )(samples_tc, keys_p, level_p, cent_p)


def kernel(samples, keys_hv, level_hv, centroid_w):
    B = samples.shape[0]
    C = centroid_w.shape[0]
    D = keys_hv.shape[1]
    pad = ((0, 0), (0, _DP - D))
    keys_p = jnp.pad(keys_hv, pad)
    level_p = jnp.pad(level_hv, pad)
    cent_p = jnp.pad(centroid_w, pad)

    m_lo, m_hi = _build_m(keys_p, level_p)
    rpt = _B_SC // _NSUB
    samples_sc = samples[:_B_SC].reshape(_NSUB, rpt, _F)
    cent_sc = cent_p.reshape(C, 2, _DH).transpose(1, 0, 2)
    sc_bag = _make_sc_bag(rpt)
    sc_out = sc_bag(m_lo, m_hi, samples_sc, cent_sc)
    sc_scores = (sc_out[0] + sc_out[1]).sum(-1).reshape(_B_SC, C)

    if _B_SC == B:
        return sc_scores
    tc_scores = _tc_slice(samples[_B_SC:], keys_p, level_p, cent_p)
    return jnp.concatenate([sc_scores, tc_scores], axis=0)


# speedup vs baseline: 1.7306x; 1.7306x over previous
import functools

import jax
import jax.numpy as jnp
from jax import lax
from jax.experimental import pallas as pl
from jax.experimental.pallas import tpu as pltpu
from jax.experimental.pallas import tpu_sc as plsc

_F = 128
_L = 100
_DP = 1024
_DQ = _DP // 4
_NSUB = 16
_NSLOT = 8
_B_SC = 64
_GROUP = 16


def _round_half_even_idx(s):
    x = (s + 1.0) * 49.5
    f0 = x.astype(jnp.int32)
    rf = x - f0.astype(jnp.float32)
    odd = (f0 & 1) == 1
    up = (rf > 0.5) | ((rf == 0.5) & odd)
    q = f0 + jnp.where(up, 1, 0)
    return jnp.minimum(jnp.maximum(q, 0), _L - 1)


def _make_sc_bag(rpt):
    mesh = plsc.VectorSubcoreMesh(
        core_axis_name="c", subcore_axis_name="s", num_cores=2,
        num_subcores=_NSUB)
    nchunk = _DQ // 16

    @functools.partial(
        pl.kernel,
        out_type=jax.ShapeDtypeStruct((4, _NSLOT, rpt, 2, 16), jnp.float32),
        mesh=mesh,
        scratch_types=[
            pltpu.VMEM((rpt, _F), jnp.float32),
            pltpu.VMEM((rpt, _F), jnp.int32),
            pltpu.VMEM((_L, _DQ), jnp.float32),
            pltpu.VMEM((_F, _DQ), jnp.float32),
            pltpu.VMEM((2, _DQ), jnp.float32),
            pltpu.VMEM((rpt, 2, 16), jnp.float32),
            pltpu.SemaphoreType.DMA,
        ],
    )
    def sc_bag(level_hbm, keys_hbm, samples_hbm, cent_hbm, out_hbm,
               samp_v, idx_v, level_v, keys_v, cent_v, out_v, sem):
        core = lax.axis_index("c")
        sub = lax.axis_index("s")
        quarter = core * 2 + sub // _NSLOT
        slot = sub % _NSLOT
        pltpu.sync_copy(samples_hbm.at[slot], samp_v)
        pltpu.sync_copy(level_hbm.at[quarter], level_v)
        pltpu.sync_copy(keys_hbm.at[quarter], keys_v)
        pltpu.sync_copy(cent_hbm.at[quarter], cent_v)

        def quant_body(i, carry):
            for c in range(_F // 16):
                idx_v[i, pl.ds(c * 16, 16)] = _round_half_even_idx(
                    samp_v[i, pl.ds(c * 16, 16)])
            return carry
        lax.fori_loop(0, rpt, quant_body, 0)

        def row_body(i, carry):
            def feat_body(c8, accs):
                lvec = idx_v[i, pl.ds(c8 * 16, 16)]
                new = list(accs)
                for lane in range(16):
                    lrow = lvec[lane]
                    f = c8 * 16 + lane
                    for k in range(nchunk):
                        new[k] = new[k] + (level_v[lrow, pl.ds(k * 16, 16)]
                                           * keys_v[f, pl.ds(k * 16, 16)])
                return tuple(new)
            acc0 = tuple(jnp.zeros((16,), jnp.float32) for _ in range(nchunk))
            accs = lax.fori_loop(0, _F // 16, feat_body, acc0)

            s0 = jnp.zeros((16,), jnp.float32)
            s1 = jnp.zeros((16,), jnp.float32)
            for k in range(nchunk):
                enc = jnp.sign(accs[k])
                s0 = s0 + enc * cent_v[0, pl.ds(k * 16, 16)]
                s1 = s1 + enc * cent_v[1, pl.ds(k * 16, 16)]
            out_v[i, 0] = s0
            out_v[i, 1] = s1
            return carry
        lax.fori_loop(0, rpt, row_body, 0)
        pltpu.sync_copy(out_v, out_hbm.at[quarter, slot])

    return sc_bag


def _tc_body(samples_ref, keys_ref, level_ref, cent_ref, out_ref):
    B = samples_ref.shape[0]
    x = (samples_ref[...] + 1.0) * (0.5 * (_L - 1))
    idx = jnp.clip(jnp.round(x), 0, _L - 1).astype(jnp.int32)
    level = level_ref[...].astype(jnp.bfloat16)
    l_iota = lax.broadcasted_iota(jnp.int32, (1, _GROUP, _L), 2)
    acc = jnp.zeros((B, _DP), jnp.float32)
    for g in range(_F // _GROUP):
        keys_g = keys_ref[pl.ds(g * _GROUP, _GROUP), :].astype(jnp.bfloat16)
        m_g = (keys_g[:, None, :] * level[None, :, :]).reshape(_GROUP * _L, _DP)
        idx_g = idx[:, g * _GROUP:(g + 1) * _GROUP]
        oh = (idx_g[:, :, None] == l_iota).astype(jnp.bfloat16).reshape(
            B, _GROUP * _L)
        acc = acc + jnp.dot(oh, m_g, preferred_element_type=jnp.float32)
    enc = jnp.sign(acc)
    out_ref[...] = lax.dot_general(
        enc, cent_ref[...], (((1,), (1,)), ((), ())),
        preferred_element_type=jnp.float32)


def _tc_slice(samples_tc, keys_p, level_p, cent_p):
    B = samples_tc.shape[0]
    return pl.pallas_call(
        _tc_body,
        out_shape=jax.ShapeDtypeStruct((B, cent_p.shape[0]), jnp.float32),
    )(samples_tc, keys_p, level_p, cent_p)


def kernel(samples, keys_hv, level_hv, centroid_w):
    B = samples.shape[0]
    C = centroid_w.shape[0]
    D = keys_hv.shape[1]
    pad = ((0, 0), (0, _DP - D))
    keys_p = jnp.pad(keys_hv, pad)
    level_p = jnp.pad(level_hv, pad)
    cent_p = jnp.pad(centroid_w, pad)

    rpt = _B_SC // _NSLOT
    samples_sc = samples[:_B_SC].reshape(_NSLOT, rpt, _F)
    level_sc = level_p.reshape(_L, 4, _DQ).transpose(1, 0, 2)
    keys_sc = keys_p.reshape(_F, 4, _DQ).transpose(1, 0, 2)
    cent_sc = cent_p.reshape(C, 4, _DQ).transpose(1, 0, 2)

    sc_bag = _make_sc_bag(rpt)
    sc_out = sc_bag(level_sc, keys_sc, samples_sc, cent_sc)
    sc_scores = sc_out.sum(axis=(0, -1)).reshape(_B_SC, C)

    if _B_SC == B:
        return sc_scores
    tc_scores = _tc_slice(samples[_B_SC:], keys_p, level_p, cent_p)
    return jnp.concatenate([sc_scores, tc_scores], axis=0)


# speedup vs baseline: 2.1955x; 1.2686x over previous
import functools

import jax
import jax.numpy as jnp
from jax import lax
from jax.experimental import pallas as pl
from jax.experimental.pallas import tpu as pltpu
from jax.experimental.pallas import tpu_sc as plsc

_F = 128
_L = 100
_DP = 1024
_DQ = _DP // 4
_NSUB = 16
_NSLOT = 8
_B_SC = 64
_GROUP = 16


def _round_half_even_idx(s):
    x = (s + 1.0) * 49.5
    f0 = x.astype(jnp.int32)
    rf = x - f0.astype(jnp.float32)
    odd = (f0 & 1) == 1
    up = (rf > 0.5) | ((rf == 0.5) & odd)
    q = f0 + jnp.where(up, 1, 0)
    return jnp.minimum(jnp.maximum(q, 0), _L - 1)


def _make_sc_bag(rpt):
    mesh = plsc.VectorSubcoreMesh(
        core_axis_name="c", subcore_axis_name="s", num_cores=2,
        num_subcores=_NSUB)
    nchunk = _DQ // 16

    @functools.partial(
        pl.kernel,
        out_type=jax.ShapeDtypeStruct((4, _NSLOT, rpt, 2, 16), jnp.float32),
        mesh=mesh,
        scratch_types=[
            pltpu.VMEM((rpt, _F), jnp.float32),
            pltpu.VMEM((rpt, _F), jnp.int32),
            pltpu.VMEM((_L, _DQ), jnp.float32),
            pltpu.VMEM((_F, _DQ), jnp.float32),
            pltpu.VMEM((2, _DQ), jnp.float32),
            pltpu.VMEM((rpt, 2, 16), jnp.float32),
            pltpu.SemaphoreType.DMA,
        ],
    )
    def sc_bag(level_hbm, keys_hbm, samples_hbm, cent_hbm, out_hbm,
               samp_v, idx_v, level_v, keys_v, cent_v, out_v, sem):
        core = lax.axis_index("c")
        sub = lax.axis_index("s")
        quarter = core * 2 + sub // _NSLOT
        slot = sub % _NSLOT
        pltpu.sync_copy(samples_hbm.at[slot], samp_v)
        pltpu.sync_copy(level_hbm.at[quarter], level_v)
        pltpu.sync_copy(keys_hbm.at[quarter], keys_v)
        pltpu.sync_copy(cent_hbm.at[quarter], cent_v)

        def quant_body(i, carry):
            for c in range(_F // 16):
                idx_v[i, pl.ds(c * 16, 16)] = _round_half_even_idx(
                    samp_v[i, pl.ds(c * 16, 16)])
            return carry
        lax.fori_loop(0, rpt, quant_body, 0)

        def row_body(i, carry):
            s0 = jnp.zeros((16,), jnp.float32)
            s1 = jnp.zeros((16,), jnp.float32)
            hc = nchunk // 2
            for h in range(2):
                off = h * hc * 16

                def feat_body(c8, accs, off=off, hc=hc):
                    lvec = idx_v[i, pl.ds(c8 * 16, 16)]
                    new = list(accs)
                    for lane in range(16):
                        lrow = lvec[lane]
                        f = c8 * 16 + lane
                        for k in range(hc):
                            new[k] = new[k] + (
                                level_v[lrow, pl.ds(off + k * 16, 16)]
                                * keys_v[f, pl.ds(off + k * 16, 16)])
                    return tuple(new)
                acc0 = tuple(jnp.zeros((16,), jnp.float32) for _ in range(hc))
                accs = lax.fori_loop(0, _F // 16, feat_body, acc0)
                for k in range(hc):
                    enc = jnp.sign(accs[k])
                    s0 = s0 + enc * cent_v[0, pl.ds(off + k * 16, 16)]
                    s1 = s1 + enc * cent_v[1, pl.ds(off + k * 16, 16)]
            out_v[i, 0] = s0
            out_v[i, 1] = s1
            return carry
        lax.fori_loop(0, rpt, row_body, 0)
        pltpu.sync_copy(out_v, out_hbm.at[quarter, slot])

    return sc_bag


def _tc_body(samples_ref, keys_ref, level_ref, cent_ref, out_ref):
    B = samples_ref.shape[0]
    x = (samples_ref[...] + 1.0) * (0.5 * (_L - 1))
    idx = jnp.clip(jnp.round(x), 0, _L - 1).astype(jnp.int32)
    level = level_ref[...].astype(jnp.bfloat16)
    l_iota = lax.broadcasted_iota(jnp.int32, (1, _GROUP, _L), 2)
    acc = jnp.zeros((B, _DP), jnp.float32)
    for g in range(_F // _GROUP):
        keys_g = keys_ref[pl.ds(g * _GROUP, _GROUP), :].astype(jnp.bfloat16)
        m_g = (keys_g[:, None, :] * level[None, :, :]).reshape(_GROUP * _L, _DP)
        idx_g = idx[:, g * _GROUP:(g + 1) * _GROUP]
        oh = (idx_g[:, :, None] == l_iota).astype(jnp.bfloat16).reshape(
            B, _GROUP * _L)
        acc = acc + jnp.dot(oh, m_g, preferred_element_type=jnp.float32)
    enc = jnp.sign(acc)
    out_ref[...] = lax.dot_general(
        enc, cent_ref[...], (((1,), (1,)), ((), ())),
        preferred_element_type=jnp.float32)


def _tc_slice(samples_tc, keys_p, level_p, cent_p):
    B = samples_tc.shape[0]
    return pl.pallas_call(
        _tc_body,
        out_shape=jax.ShapeDtypeStruct((B, cent_p.shape[0]), jnp.float32),
    )(samples_tc, keys_p, level_p, cent_p)


def kernel(samples, keys_hv, level_hv, centroid_w):
    B = samples.shape[0]
    C = centroid_w.shape[0]
    D = keys_hv.shape[1]
    pad = ((0, 0), (0, _DP - D))
    keys_p = jnp.pad(keys_hv, pad)
    level_p = jnp.pad(level_hv, pad)
    cent_p = jnp.pad(centroid_w, pad)

    rpt = _B_SC // _NSLOT
    samples_sc = samples[:_B_SC].reshape(_NSLOT, rpt, _F)
    level_sc = level_p.reshape(_L, 4, _DQ).transpose(1, 0, 2)
    keys_sc = keys_p.reshape(_F, 4, _DQ).transpose(1, 0, 2)
    cent_sc = cent_p.reshape(C, 4, _DQ).transpose(1, 0, 2)

    sc_bag = _make_sc_bag(rpt)
    if _B_SC == B:
        sc_out = sc_bag(level_sc, keys_sc, samples_sc, cent_sc)
        return sc_out.sum(axis=(0, -1)).reshape(_B_SC, C)
    tc_scores = _tc_slice(samples[_B_SC:], keys_p, level_p, cent_p)
    sc_out = sc_bag(level_sc, keys_sc, samples_sc, cent_sc)
    sc_scores = sc_out.sum(axis=(0, -1)).reshape(_B_SC, C)
    return jnp.concatenate([sc_scores, tc_scores], axis=0)


# speedup vs baseline: 2.9481x; 1.3428x over previous
import functools

import jax
import jax.numpy as jnp
from jax import lax
from jax.experimental import pallas as pl
from jax.experimental.pallas import tpu as pltpu
from jax.experimental.pallas import tpu_sc as plsc

_F = 128
_L = 100
_DP = 1024
_DQ = _DP // 4
_NSUB = 16
_NSLOT = 8
_B_SC = 64
_GROUP = 16


def _round_half_even_idx(s):
    x = (s + 1.0) * 49.5
    f0 = x.astype(jnp.int32)
    rf = x - f0.astype(jnp.float32)
    odd = (f0 & 1) == 1
    up = (rf > 0.5) | ((rf == 0.5) & odd)
    q = f0 + jnp.where(up, 1, 0)
    return jnp.minimum(jnp.maximum(q, 0), _L - 1)


def _make_sc_bag(rpt):
    mesh = plsc.VectorSubcoreMesh(
        core_axis_name="c", subcore_axis_name="s", num_cores=2,
        num_subcores=_NSUB)
    nchunk = _DQ // 16

    @functools.partial(
        pl.kernel,
        out_type=jax.ShapeDtypeStruct((4, _NSLOT, rpt, 2, 16), jnp.float32),
        mesh=mesh,
        scratch_types=[
            pltpu.VMEM((rpt, _F), jnp.float32),
            pltpu.VMEM((rpt, _F), jnp.int32),
            pltpu.VMEM((_L, _DQ), jnp.float32),
            pltpu.VMEM((_F, _DQ), jnp.float32),
            pltpu.VMEM((2, _DQ), jnp.float32),
            pltpu.VMEM((rpt, 2, 16), jnp.float32),
            pltpu.SemaphoreType.DMA,
        ],
    )
    def sc_bag(level_hbm, keys_hbm, samples_hbm, cent_hbm, out_hbm,
               samp_v, idx_v, level_v, keys_v, cent_v, out_v, sem):
        core = lax.axis_index("c")
        sub = lax.axis_index("s")
        quarter = core * 2 + sub // _NSLOT
        slot = sub % _NSLOT
        pltpu.sync_copy(samples_hbm.at[slot], samp_v)
        pltpu.sync_copy(level_hbm.at[quarter], level_v)
        pltpu.sync_copy(keys_hbm.at[quarter], keys_v)
        pltpu.sync_copy(cent_hbm.at[quarter], cent_v)

        def quant_body(i, carry):
            for c in range(_F // 16):
                idx_v[i, pl.ds(c * 16, 16)] = _round_half_even_idx(
                    samp_v[i, pl.ds(c * 16, 16)])
            return carry
        lax.fori_loop(0, rpt, quant_body, 0)

        def row_body(i, carry):
            s0 = jnp.zeros((16,), jnp.float32)
            s1 = jnp.zeros((16,), jnp.float32)
            hc = nchunk // 4
            for h in range(4):
                off = h * hc * 16

                def feat_body(c8, accs, off=off, hc=hc):
                    lvec = idx_v[i, pl.ds(c8 * 16, 16)]
                    new = list(accs)
                    for lane in range(16):
                        lrow = lvec[lane]
                        f = c8 * 16 + lane
                        for k in range(hc):
                            new[k] = new[k] + (
                                level_v[lrow, pl.ds(off + k * 16, 16)]
                                * keys_v[f, pl.ds(off + k * 16, 16)])
                    return tuple(new)
                acc0 = tuple(jnp.zeros((16,), jnp.float32) for _ in range(hc))
                accs = lax.fori_loop(0, _F // 16, feat_body, acc0)
                for k in range(hc):
                    enc = jnp.sign(accs[k])
                    s0 = s0 + enc * cent_v[0, pl.ds(off + k * 16, 16)]
                    s1 = s1 + enc * cent_v[1, pl.ds(off + k * 16, 16)]
            out_v[i, 0] = s0
            out_v[i, 1] = s1
            return carry
        lax.fori_loop(0, rpt, row_body, 0)
        pltpu.sync_copy(out_v, out_hbm.at[quarter, slot])

    return sc_bag


def _tc_body(samples_ref, keys_ref, level_ref, cent_ref, out_ref):
    B = samples_ref.shape[0]
    x = (samples_ref[...] + 1.0) * (0.5 * (_L - 1))
    idx = jnp.clip(jnp.round(x), 0, _L - 1).astype(jnp.int32)
    level = level_ref[...].astype(jnp.bfloat16)
    l_iota = lax.broadcasted_iota(jnp.int32, (1, _GROUP, _L), 2)
    acc = jnp.zeros((B, _DP), jnp.float32)
    for g in range(_F // _GROUP):
        keys_g = keys_ref[pl.ds(g * _GROUP, _GROUP), :].astype(jnp.bfloat16)
        m_g = (keys_g[:, None, :] * level[None, :, :]).reshape(_GROUP * _L, _DP)
        idx_g = idx[:, g * _GROUP:(g + 1) * _GROUP]
        oh = (idx_g[:, :, None] == l_iota).astype(jnp.bfloat16).reshape(
            B, _GROUP * _L)
        acc = acc + jnp.dot(oh, m_g, preferred_element_type=jnp.float32)
    enc = jnp.sign(acc)
    out_ref[...] = lax.dot_general(
        enc, cent_ref[...], (((1,), (1,)), ((), ())),
        preferred_element_type=jnp.float32)


def _tc_slice(samples_tc, keys_p, level_p, cent_p):
    B = samples_tc.shape[0]
    return pl.pallas_call(
        _tc_body,
        out_shape=jax.ShapeDtypeStruct((B, cent_p.shape[0]), jnp.float32),
    )(samples_tc, keys_p, level_p, cent_p)


def kernel(samples, keys_hv, level_hv, centroid_w):
    B = samples.shape[0]
    C = centroid_w.shape[0]
    D = keys_hv.shape[1]
    pad = ((0, 0), (0, _DP - D))
    keys_p = jnp.pad(keys_hv, pad)
    level_p = jnp.pad(level_hv, pad)
    cent_p = jnp.pad(centroid_w, pad)

    rpt = _B_SC // _NSLOT
    samples_sc = samples[:_B_SC].reshape(_NSLOT, rpt, _F)
    level_sc = level_p.reshape(_L, 4, _DQ).transpose(1, 0, 2)
    keys_sc = keys_p.reshape(_F, 4, _DQ).transpose(1, 0, 2)
    cent_sc = cent_p.reshape(C, 4, _DQ).transpose(1, 0, 2)

    sc_bag = _make_sc_bag(rpt)
    if _B_SC == B:
        sc_out = sc_bag(level_sc, keys_sc, samples_sc, cent_sc)
        return sc_out.sum(axis=(0, -1)).reshape(_B_SC, C)
    tc_scores = _tc_slice(samples[_B_SC:], keys_p, level_p, cent_p)
    sc_out = sc_bag(level_sc, keys_sc, samples_sc, cent_sc)
    sc_scores = sc_out.sum(axis=(0, -1)).reshape(_B_SC, C)
    return jnp.concatenate([sc_scores, tc_scores], axis=0)


# speedup vs baseline: 3.3148x; 1.1244x over previous
import functools

import jax
import jax.numpy as jnp
from jax import lax
from jax.experimental import pallas as pl
from jax.experimental.pallas import tpu as pltpu
from jax.experimental.pallas import tpu_sc as plsc

_F = 128
_L = 100
_DP = 1024
_DQ = _DP // 4
_NSUB = 16
_NSLOT = 8
_B_SC = 64
_GROUP = 16


def _round_half_even_idx(s):
    x = (s + 1.0) * 49.5
    f0 = x.astype(jnp.int32)
    rf = x - f0.astype(jnp.float32)
    odd = (f0 & 1) == 1
    up = (rf > 0.5) | ((rf == 0.5) & odd)
    q = f0 + jnp.where(up, 1, 0)
    return jnp.minimum(jnp.maximum(q, 0), _L - 1)


def _make_sc_bag(rpt):
    mesh = plsc.VectorSubcoreMesh(
        core_axis_name="c", subcore_axis_name="s", num_cores=2,
        num_subcores=_NSUB)
    nchunk = _DQ // 16

    @functools.partial(
        pl.kernel,
        out_type=jax.ShapeDtypeStruct((4, _NSLOT, rpt, 2, 16), jnp.float32),
        mesh=mesh,
        scratch_types=[
            pltpu.VMEM((rpt, _F), jnp.float32),
            pltpu.VMEM((rpt, _F), jnp.int32),
            pltpu.VMEM((_L, _DQ), jnp.float32),
            pltpu.VMEM((_F, _DQ), jnp.float32),
            pltpu.VMEM((2, _DQ), jnp.float32),
            pltpu.VMEM((rpt, 2, 16), jnp.float32),
            pltpu.SemaphoreType.DMA,
        ],
    )
    def sc_bag(level_hbm, keys_hbm, samples_hbm, cent_hbm, out_hbm,
               samp_v, idx_v, level_v, keys_v, cent_v, out_v, sem):
        core = lax.axis_index("c")
        sub = lax.axis_index("s")
        quarter = core * 2 + sub // _NSLOT
        slot = sub % _NSLOT
        pltpu.sync_copy(samples_hbm.at[slot], samp_v)
        pltpu.sync_copy(level_hbm.at[quarter], level_v)
        pltpu.sync_copy(keys_hbm.at[quarter], keys_v)
        pltpu.sync_copy(cent_hbm.at[quarter], cent_v)

        def quant_body(i, carry):
            for c in range(_F // 16):
                idx_v[i, pl.ds(c * 16, 16)] = _round_half_even_idx(
                    samp_v[i, pl.ds(c * 16, 16)])
            return carry
        lax.fori_loop(0, rpt, quant_body, 0)

        def row_body(i2, carry):
            ia = i2 * 2
            ib = ia + 1
            s0a = jnp.zeros((16,), jnp.float32)
            s1a = jnp.zeros((16,), jnp.float32)
            s0b = jnp.zeros((16,), jnp.float32)
            s1b = jnp.zeros((16,), jnp.float32)
            hc = nchunk // 4
            for h in range(4):
                off = h * hc * 16

                def feat_body(c8, accs, off=off, hc=hc):
                    lveca = idx_v[ia, pl.ds(c8 * 16, 16)]
                    lvecb = idx_v[ib, pl.ds(c8 * 16, 16)]
                    acca = list(accs[0])
                    accb = list(accs[1])
                    for lane in range(16):
                        la = lveca[lane]
                        lb = lvecb[lane]
                        f = c8 * 16 + lane
                        for k in range(hc):
                            kv = keys_v[f, pl.ds(off + k * 16, 16)]
                            acca[k] = acca[k] + (
                                level_v[la, pl.ds(off + k * 16, 16)] * kv)
                            accb[k] = accb[k] + (
                                level_v[lb, pl.ds(off + k * 16, 16)] * kv)
                    return (tuple(acca), tuple(accb))
                z16 = tuple(jnp.zeros((16,), jnp.float32) for _ in range(hc))
                acca, accb = lax.fori_loop(0, _F // 16, feat_body, (z16, z16))
                for k in range(hc):
                    enca = jnp.sign(acca[k])
                    encb = jnp.sign(accb[k])
                    c0 = cent_v[0, pl.ds(off + k * 16, 16)]
                    c1 = cent_v[1, pl.ds(off + k * 16, 16)]
                    s0a = s0a + enca * c0
                    s1a = s1a + enca * c1
                    s0b = s0b + encb * c0
                    s1b = s1b + encb * c1
            out_v[ia, 0] = s0a
            out_v[ia, 1] = s1a
            out_v[ib, 0] = s0b
            out_v[ib, 1] = s1b
            return carry
        lax.fori_loop(0, rpt // 2, row_body, 0)
        pltpu.sync_copy(out_v, out_hbm.at[quarter, slot])

    return sc_bag


def _tc_body(samples_ref, keys_ref, level_ref, cent_ref, out_ref):
    B = samples_ref.shape[0]
    x = (samples_ref[...] + 1.0) * (0.5 * (_L - 1))
    idx = jnp.clip(jnp.round(x), 0, _L - 1).astype(jnp.int32)
    level = level_ref[...].astype(jnp.bfloat16)
    l_iota = lax.broadcasted_iota(jnp.int32, (1, _GROUP, _L), 2)
    acc = jnp.zeros((B, _DP), jnp.float32)
    for g in range(_F // _GROUP):
        keys_g = keys_ref[pl.ds(g * _GROUP, _GROUP), :].astype(jnp.bfloat16)
        m_g = (keys_g[:, None, :] * level[None, :, :]).reshape(_GROUP * _L, _DP)
        idx_g = idx[:, g * _GROUP:(g + 1) * _GROUP]
        oh = (idx_g[:, :, None] == l_iota).astype(jnp.bfloat16).reshape(
            B, _GROUP * _L)
        acc = acc + jnp.dot(oh, m_g, preferred_element_type=jnp.float32)
    enc = jnp.sign(acc)
    out_ref[...] = lax.dot_general(
        enc, cent_ref[...], (((1,), (1,)), ((), ())),
        preferred_element_type=jnp.float32)


def _tc_slice(samples_tc, keys_p, level_p, cent_p):
    B = samples_tc.shape[0]
    return pl.pallas_call(
        _tc_body,
        out_shape=jax.ShapeDtypeStruct((B, cent_p.shape[0]), jnp.float32),
    )(samples_tc, keys_p, level_p, cent_p)


def kernel(samples, keys_hv, level_hv, centroid_w):
    B = samples.shape[0]
    C = centroid_w.shape[0]
    D = keys_hv.shape[1]
    pad = ((0, 0), (0, _DP - D))
    keys_p = jnp.pad(keys_hv, pad)
    level_p = jnp.pad(level_hv, pad)
    cent_p = jnp.pad(centroid_w, pad)

    rpt = _B_SC // _NSLOT
    samples_sc = samples[:_B_SC].reshape(_NSLOT, rpt, _F)
    level_sc = level_p.reshape(_L, 4, _DQ).transpose(1, 0, 2)
    keys_sc = keys_p.reshape(_F, 4, _DQ).transpose(1, 0, 2)
    cent_sc = cent_p.reshape(C, 4, _DQ).transpose(1, 0, 2)

    sc_bag = _make_sc_bag(rpt)
    if _B_SC == B:
        sc_out = sc_bag(level_sc, keys_sc, samples_sc, cent_sc)
        return sc_out.sum(axis=(0, -1)).reshape(_B_SC, C)
    tc_scores = _tc_slice(samples[_B_SC:], keys_p, level_p, cent_p)
    sc_out = sc_bag(level_sc, keys_sc, samples_sc, cent_sc)
    sc_scores = sc_out.sum(axis=(0, -1)).reshape(_B_SC, C)
    return jnp.concatenate([sc_scores, tc_scores], axis=0)


# speedup vs baseline: 4.3723x; 1.3190x over previous
import functools

import jax
import jax.numpy as jnp
from jax import lax
from jax.experimental import pallas as pl
from jax.experimental.pallas import tpu as pltpu
from jax.experimental.pallas import tpu_sc as plsc

_F = 128
_L = 100
_DP = 1024
_DQ = _DP // 4
_NSUB = 16
_NSLOT = 8
_B_SC = 32
_GROUP = 16


def _round_half_even_idx(s):
    x = (s + 1.0) * 49.5
    f0 = x.astype(jnp.int32)
    rf = x - f0.astype(jnp.float32)
    odd = (f0 & 1) == 1
    up = (rf > 0.5) | ((rf == 0.5) & odd)
    q = f0 + jnp.where(up, 1, 0)
    return jnp.minimum(jnp.maximum(q, 0), _L - 1)


def _make_sc_bag(rpt):
    mesh = plsc.VectorSubcoreMesh(
        core_axis_name="c", subcore_axis_name="s", num_cores=2,
        num_subcores=_NSUB)
    nchunk = _DQ // 16

    @functools.partial(
        pl.kernel,
        out_type=jax.ShapeDtypeStruct((4, _NSLOT, rpt, 2, 16), jnp.float32),
        mesh=mesh,
        scratch_types=[
            pltpu.VMEM((rpt, _F), jnp.float32),
            pltpu.VMEM((rpt, _F), jnp.int32),
            pltpu.VMEM((_L, _DQ), jnp.float32),
            pltpu.VMEM((_F, _DQ), jnp.float32),
            pltpu.VMEM((2, _DQ), jnp.float32),
            pltpu.VMEM((rpt, 2, 16), jnp.float32),
            pltpu.SemaphoreType.DMA,
        ],
    )
    def sc_bag(level_hbm, keys_hbm, samples_hbm, cent_hbm, out_hbm,
               samp_v, idx_v, level_v, keys_v, cent_v, out_v, sem):
        core = lax.axis_index("c")
        sub = lax.axis_index("s")
        quarter = core * 2 + sub // _NSLOT
        slot = sub % _NSLOT
        pltpu.sync_copy(samples_hbm.at[slot], samp_v)
        pltpu.sync_copy(level_hbm.at[quarter], level_v)
        pltpu.sync_copy(keys_hbm.at[quarter], keys_v)
        pltpu.sync_copy(cent_hbm.at[quarter], cent_v)

        def quant_body(i, carry):
            for c in range(_F // 16):
                idx_v[i, pl.ds(c * 16, 16)] = _round_half_even_idx(
                    samp_v[i, pl.ds(c * 16, 16)])
            return carry
        lax.fori_loop(0, rpt, quant_body, 0)

        def row_body(i2, carry):
            ia = i2 * 2
            ib = ia + 1
            s0a = jnp.zeros((16,), jnp.float32)
            s1a = jnp.zeros((16,), jnp.float32)
            s0b = jnp.zeros((16,), jnp.float32)
            s1b = jnp.zeros((16,), jnp.float32)
            hc = nchunk // 4
            for h in range(4):
                off = h * hc * 16

                def feat_body(c8, accs, off=off, hc=hc):
                    lveca = idx_v[ia, pl.ds(c8 * 16, 16)]
                    lvecb = idx_v[ib, pl.ds(c8 * 16, 16)]
                    acca = list(accs[0])
                    accb = list(accs[1])
                    for lane in range(16):
                        la = lveca[lane]
                        lb = lvecb[lane]
                        f = c8 * 16 + lane
                        for k in range(hc):
                            kv = keys_v[f, pl.ds(off + k * 16, 16)]
                            acca[k] = acca[k] + (
                                level_v[la, pl.ds(off + k * 16, 16)] * kv)
                            accb[k] = accb[k] + (
                                level_v[lb, pl.ds(off + k * 16, 16)] * kv)
                    return (tuple(acca), tuple(accb))
                z16 = tuple(jnp.zeros((16,), jnp.float32) for _ in range(hc))
                acca, accb = lax.fori_loop(0, _F // 16, feat_body, (z16, z16))
                for k in range(hc):
                    enca = jnp.sign(acca[k])
                    encb = jnp.sign(accb[k])
                    c0 = cent_v[0, pl.ds(off + k * 16, 16)]
                    c1 = cent_v[1, pl.ds(off + k * 16, 16)]
                    s0a = s0a + enca * c0
                    s1a = s1a + enca * c1
                    s0b = s0b + encb * c0
                    s1b = s1b + encb * c1
            out_v[ia, 0] = s0a
            out_v[ia, 1] = s1a
            out_v[ib, 0] = s0b
            out_v[ib, 1] = s1b
            return carry
        lax.fori_loop(0, rpt // 2, row_body, 0)
        pltpu.sync_copy(out_v, out_hbm.at[quarter, slot])

    return sc_bag


def _tc_body(samples_ref, keys_ref, level_ref, cent_ref, out_ref):
    B = samples_ref.shape[0]
    x = (samples_ref[...] + 1.0) * (0.5 * (_L - 1))
    idx = jnp.clip(jnp.round(x), 0, _L - 1).astype(jnp.int32)
    level = level_ref[...].astype(jnp.bfloat16)
    l_iota = lax.broadcasted_iota(jnp.int32, (1, _GROUP, _L), 2)
    acc = jnp.zeros((B, _DP), jnp.float32)
    for g in range(_F // _GROUP):
        keys_g = keys_ref[pl.ds(g * _GROUP, _GROUP), :].astype(jnp.bfloat16)
        m_g = (keys_g[:, None, :] * level[None, :, :]).reshape(_GROUP * _L, _DP)
        idx_g = idx[:, g * _GROUP:(g + 1) * _GROUP]
        oh = (idx_g[:, :, None] == l_iota).astype(jnp.bfloat16).reshape(
            B, _GROUP * _L)
        acc = acc + jnp.dot(oh, m_g, preferred_element_type=jnp.float32)
    enc = jnp.sign(acc)
    out_ref[...] = lax.dot_general(
        enc, cent_ref[...], (((1,), (1,)), ((), ())),
        preferred_element_type=jnp.float32)


def _tc_slice(samples_tc, keys_p, level_p, cent_p):
    B = samples_tc.shape[0]
    return pl.pallas_call(
        _tc_body,
        out_shape=jax.ShapeDtypeStruct((B, cent_p.shape[0]), jnp.float32),
    )(samples_tc, keys_p, level_p, cent_p)


def kernel(samples, keys_hv, level_hv, centroid_w):
    B = samples.shape[0]
    C = centroid_w.shape[0]
    D = keys_hv.shape[1]
    pad = ((0, 0), (0, _DP - D))
    keys_p = jnp.pad(keys_hv, pad)
    level_p = jnp.pad(level_hv, pad)
    cent_p = jnp.pad(centroid_w, pad)

    rpt = _B_SC // _NSLOT
    samples_sc = samples[:_B_SC].reshape(_NSLOT, rpt, _F)
    level_sc = level_p.reshape(_L, 4, _DQ).transpose(1, 0, 2)
    keys_sc = keys_p.reshape(_F, 4, _DQ).transpose(1, 0, 2)
    cent_sc = cent_p.reshape(C, 4, _DQ).transpose(1, 0, 2)

    sc_bag = _make_sc_bag(rpt)
    if _B_SC == B:
        sc_out = sc_bag(level_sc, keys_sc, samples_sc, cent_sc)
        return sc_out.sum(axis=(0, -1)).reshape(_B_SC, C)
    tc_scores = _tc_slice(samples[_B_SC:], keys_p, level_p, cent_p)
    sc_out = sc_bag(level_sc, keys_sc, samples_sc, cent_sc)
    sc_scores = sc_out.sum(axis=(0, -1)).reshape(_B_SC, C)
    return jnp.concatenate([sc_scores, tc_scores], axis=0)


# speedup vs baseline: 4.4363x; 1.0146x over previous
import functools

import jax
import jax.numpy as jnp
from jax import lax
from jax.experimental import pallas as pl
from jax.experimental.pallas import tpu as pltpu
from jax.experimental.pallas import tpu_sc as plsc

_F = 128
_L = 100
_DP = 1024
_DQ = _DP // 4
_NSUB = 16
_NSLOT = 8
_B_SC = 32
_GROUP = 16


def _round_half_even_idx(s):
    x = (s + 1.0) * 49.5
    f0 = x.astype(jnp.int32)
    rf = x - f0.astype(jnp.float32)
    odd = (f0 & 1) == 1
    up = (rf > 0.5) | ((rf == 0.5) & odd)
    q = f0 + jnp.where(up, 1, 0)
    return jnp.minimum(jnp.maximum(q, 0), _L - 1)


def _make_sc_bag(rpt):
    mesh = plsc.VectorSubcoreMesh(
        core_axis_name="c", subcore_axis_name="s", num_cores=2,
        num_subcores=_NSUB)
    nchunk = _DQ // 16

    @functools.partial(
        pl.kernel,
        out_type=jax.ShapeDtypeStruct((4, _NSLOT, rpt, 2, 16), jnp.float32),
        mesh=mesh,
        scratch_types=[
            pltpu.VMEM((rpt, _F), jnp.float32),
            pltpu.VMEM((rpt, _F), jnp.int32),
            pltpu.VMEM((_L, _DQ), jnp.float32),
            pltpu.VMEM((_F, _DQ), jnp.float32),
            pltpu.VMEM((2, _DQ), jnp.float32),
            pltpu.VMEM((rpt, 2, 16), jnp.float32),
            pltpu.SemaphoreType.DMA,
        ],
    )
    def sc_bag(level_hbm, keys_hbm, samples_hbm, cent_hbm, out_hbm,
               samp_v, idx_v, level_v, keys_v, cent_v, out_v, sem):
        core = lax.axis_index("c")
        sub = lax.axis_index("s")
        quarter = core * 2 + sub // _NSLOT
        slot = sub % _NSLOT
        dq0 = quarter * _DQ
        pltpu.sync_copy(samples_hbm.at[slot], samp_v)
        pltpu.sync_copy(level_hbm.at[:, pl.ds(dq0, _DQ)], level_v)
        pltpu.sync_copy(keys_hbm.at[:, pl.ds(dq0, _DQ)], keys_v)
        pltpu.sync_copy(cent_hbm.at[:, pl.ds(dq0, _DQ)], cent_v)

        def quant_body(i, carry):
            for c in range(_F // 16):
                idx_v[i, pl.ds(c * 16, 16)] = _round_half_even_idx(
                    samp_v[i, pl.ds(c * 16, 16)])
            return carry
        lax.fori_loop(0, rpt, quant_body, 0)

        def row_body(i2, carry):
            ia = i2 * 2
            ib = ia + 1
            s0a = jnp.zeros((16,), jnp.float32)
            s1a = jnp.zeros((16,), jnp.float32)
            s0b = jnp.zeros((16,), jnp.float32)
            s1b = jnp.zeros((16,), jnp.float32)
            hc = nchunk // 4
            for h in range(4):
                off = h * hc * 16

                def feat_body(c8, accs, off=off, hc=hc):
                    lveca = idx_v[ia, pl.ds(c8 * 16, 16)]
                    lvecb = idx_v[ib, pl.ds(c8 * 16, 16)]
                    acca = list(accs[0])
                    accb = list(accs[1])
                    for lane in range(16):
                        la = lveca[lane]
                        lb = lvecb[lane]
                        f = c8 * 16 + lane
                        for k in range(hc):
                            kv = keys_v[f, pl.ds(off + k * 16, 16)]
                            acca[k] = acca[k] + (
                                level_v[la, pl.ds(off + k * 16, 16)] * kv)
                            accb[k] = accb[k] + (
                                level_v[lb, pl.ds(off + k * 16, 16)] * kv)
                    return (tuple(acca), tuple(accb))
                z16 = tuple(jnp.zeros((16,), jnp.float32) for _ in range(hc))
                acca, accb = lax.fori_loop(0, _F // 16, feat_body, (z16, z16))
                for k in range(hc):
                    enca = jnp.sign(acca[k])
                    encb = jnp.sign(accb[k])
                    c0 = cent_v[0, pl.ds(off + k * 16, 16)]
                    c1 = cent_v[1, pl.ds(off + k * 16, 16)]
                    s0a = s0a + enca * c0
                    s1a = s1a + enca * c1
                    s0b = s0b + encb * c0
                    s1b = s1b + encb * c1
            out_v[ia, 0] = s0a
            out_v[ia, 1] = s1a
            out_v[ib, 0] = s0b
            out_v[ib, 1] = s1b
            return carry
        lax.fori_loop(0, rpt // 2, row_body, 0)
        pltpu.sync_copy(out_v, out_hbm.at[quarter, slot])

    return sc_bag


def _tc_body(samples_ref, keys_ref, level_ref, cent_ref, out_ref):
    B = samples_ref.shape[0]
    x = (samples_ref[...] + 1.0) * (0.5 * (_L - 1))
    idx = jnp.clip(jnp.round(x), 0, _L - 1).astype(jnp.int32)
    level = level_ref[...].astype(jnp.bfloat16)
    l_iota = lax.broadcasted_iota(jnp.int32, (1, _GROUP, _L), 2)
    acc = jnp.zeros((B, _DP), jnp.float32)
    for g in range(_F // _GROUP):
        keys_g = keys_ref[pl.ds(g * _GROUP, _GROUP), :].astype(jnp.bfloat16)
        m_g = (keys_g[:, None, :] * level[None, :, :]).reshape(_GROUP * _L, _DP)
        idx_g = idx[:, g * _GROUP:(g + 1) * _GROUP]
        oh = (idx_g[:, :, None] == l_iota).astype(jnp.bfloat16).reshape(
            B, _GROUP * _L)
        acc = acc + jnp.dot(oh, m_g, preferred_element_type=jnp.float32)
    enc = jnp.sign(acc)
    out_ref[...] = lax.dot_general(
        enc, cent_ref[...], (((1,), (1,)), ((), ())),
        preferred_element_type=jnp.float32)


def _tc_slice(samples_tc, keys_p, level_p, cent_p):
    B = samples_tc.shape[0]
    return pl.pallas_call(
        _tc_body,
        out_shape=jax.ShapeDtypeStruct((B, cent_p.shape[0]), jnp.float32),
    )(samples_tc, keys_p, level_p, cent_p)


def kernel(samples, keys_hv, level_hv, centroid_w):
    B = samples.shape[0]
    C = centroid_w.shape[0]
    D = keys_hv.shape[1]
    pad = ((0, 0), (0, _DP - D))
    keys_p = jnp.pad(keys_hv, pad)
    level_p = jnp.pad(level_hv, pad)
    cent_p = jnp.pad(centroid_w, pad)

    rpt = _B_SC // _NSLOT
    samples_sc = samples[:_B_SC].reshape(_NSLOT, rpt, _F)
    sc_bag = _make_sc_bag(rpt)
    if _B_SC == B:
        sc_out = sc_bag(level_p, keys_p, samples_sc, cent_p)
        return sc_out.sum(axis=(0, -1)).reshape(_B_SC, C)
    tc_scores = _tc_slice(samples[_B_SC:], keys_p, level_p, cent_p)
    sc_out = sc_bag(level_p, keys_p, samples_sc, cent_p)
    sc_scores = sc_out.sum(axis=(0, -1)).reshape(_B_SC, C)
    return jnp.concatenate([sc_scores, tc_scores], axis=0)
